# Initial kernel scaffold; baseline (speedup 1.0000x reference)
import jax
import jax.numpy as jnp
from jax.experimental import pallas as pl


def kernel(antenna_indices, embeddings):
    raise NotImplementedError("write your pallas kernel here")


# speedup vs baseline: 4.1530x; 4.1530x over previous
import functools

import jax
import jax.numpy as jnp
from jax import lax
from jax.experimental import pallas as pl
from jax.experimental.pallas import tpu as pltpu
from jax.experimental.pallas import tpu_sc as plsc

BATCH = 4096
NUM_ANT = 200
DIM = 64
B = BATCH * NUM_ANT
NC, NS = 2, 16
NW = NC * NS
RPS = 128
K = 8
BLOCK = K * RPS
B_PER_W = B // NW
NBLK = B_PER_W // BLOCK

@functools.cache
def _build():
    mesh = plsc.VectorSubcoreMesh(core_axis_name="c", subcore_axis_name="s")

    @functools.partial(
        pl.kernel,
        out_type=jax.ShapeDtypeStruct((B, DIM), jnp.float32),
        mesh=mesh,
        scratch_types=[
            pltpu.VMEM((K, RPS), jnp.int32),
            pltpu.VMEM((BLOCK, DIM), jnp.float32),
            pltpu.SemaphoreType.DMA,
        ],
        compiler_params=pltpu.CompilerParams(use_tc_tiling_on_sc=False),
    )
    def _gather(table_hbm, idx_hbm, out_hbm, idx_v, rows_v, sem):
        wid = lax.axis_index("s") * NC + lax.axis_index("c")
        row0 = wid * (B_PER_W // RPS)

        def body(g, carry):
            r = row0 + g * K
            pltpu.sync_copy(idx_hbm.at[pl.ds(r, K)], idx_v)
            cps = [
                pltpu.async_copy(
                    table_hbm.at[idx_v.at[j]],
                    rows_v.at[pl.ds(j * RPS, RPS)],
                    sem,
                )
                for j in range(K)
            ]
            for c in cps:
                c.wait()
            pltpu.sync_copy(rows_v, out_hbm.at[pl.ds(r * RPS, BLOCK)])
            return carry

        lax.fori_loop(0, NBLK, body, 0)

    return _gather


def kernel(antenna_indices, embeddings):
    idx = antenna_indices.astype(jnp.int32).reshape(B // RPS, RPS)
    out = _build()(embeddings, idx)
    return out.reshape(BATCH, NUM_ANT, DIM)


# speedup vs baseline: 4.2790x; 1.0303x over previous
import functools

import jax
import jax.numpy as jnp
from jax import lax
from jax.experimental import pallas as pl
from jax.experimental.pallas import tpu as pltpu
from jax.experimental.pallas import tpu_sc as plsc

BATCH = 4096
NUM_ANT = 200
DIM = 64
B = BATCH * NUM_ANT
NC, NS = 2, 16
NW = NC * NS
RPS = 128
K = 5
BLOCK = K * RPS
B_PER_W = B // NW
NBLK = B_PER_W // BLOCK
IDX_ROWS = B_PER_W // RPS


@functools.cache
def _build():
    mesh = plsc.VectorSubcoreMesh(core_axis_name="c", subcore_axis_name="s")

    @functools.partial(
        pl.kernel,
        out_type=jax.ShapeDtypeStruct((B, DIM), jnp.float32),
        mesh=mesh,
        scratch_types=[
            pltpu.VMEM((IDX_ROWS, RPS), jnp.int32),
            pltpu.VMEM((BLOCK, DIM), jnp.float32),
            pltpu.VMEM((BLOCK, DIM), jnp.float32),
            pltpu.SemaphoreType.DMA,
            pltpu.SemaphoreType.DMA,
            pltpu.SemaphoreType.DMA,
            pltpu.SemaphoreType.DMA,
        ],
        compiler_params=pltpu.CompilerParams(use_tc_tiling_on_sc=False),
    )
    def _gather(table_hbm, idx_hbm, out_hbm, idx_v, rows0, rows1,
                sg0, sg1, so0, so1):
        rows_v = (rows0, rows1)
        sem_gat = (sg0, sg1)
        sem_out = (so0, so1)
        wid = lax.axis_index("s") * NC + lax.axis_index("c")
        row0 = wid * IDX_ROWS
        out0 = wid * B_PER_W

        def fire(g, b):
            for j in range(K):
                pltpu.async_copy(
                    table_hbm.at[idx_v.at[g * K + j]],
                    rows_v[b].at[pl.ds(j * RPS, RPS)],
                    sem_gat[b],
                )

        def wait_gat(b):
            pltpu.make_async_copy(
                out_hbm.at[pl.ds(0, BLOCK)], rows_v[b], sem_gat[b]
            ).wait()

        def writeback(g, b):
            pltpu.async_copy(
                rows_v[b], out_hbm.at[pl.ds(out0 + g * BLOCK, BLOCK)],
                sem_out[b],
            )

        def wait_out(b):
            pltpu.make_async_copy(
                rows_v[b], out_hbm.at[pl.ds(0, BLOCK)], sem_out[b]
            ).wait()

        pltpu.sync_copy(idx_hbm.at[pl.ds(row0, IDX_ROWS)], idx_v)

        fire(0, 0)
        fire(1, 1)
        wait_gat(0)
        writeback(0, 0)

        def body(t, carry):
            g = 2 + 2 * t
            wait_out(0)
            fire(g, 0)
            wait_gat(1)
            writeback(g - 1, 1)
            wait_out(1)
            fire(g + 1, 1)
            wait_gat(0)
            writeback(g, 0)
            return carry

        lax.fori_loop(0, (NBLK - 2) // 2, body, 0)

        wait_gat(1)
        writeback(NBLK - 1, 1)
        wait_out(0)
        wait_out(1)

    return _gather


def kernel(antenna_indices, embeddings):
    idx = antenna_indices.astype(jnp.int32).reshape(B // RPS, RPS)
    out = _build()(embeddings, idx)
    return out.reshape(BATCH, NUM_ANT, DIM)
